# initial kernel scaffold (unmeasured)
import jax
import jax.numpy as jnp
from jax import lax
from jax.experimental import pallas as pl
from jax.experimental.pallas import tpu as pltpu


def kernel(
    x,
):
    def body(*refs):
        pass

    out_shape = jax.ShapeDtypeStruct(..., jnp.float32)
    return pl.pallas_call(body, out_shape=out_shape)(...)


# baseline (device time: 82806 ns/iter reference)
import jax
import jax.numpy as jnp
from jax import lax
from jax.experimental import pallas as pl
from jax.experimental.pallas import tpu as pltpu

B = 384
PAD = 8
DEBUG_NO_RDMA = False


def kernel(x):
    m, n = x.shape
    C = m // B

    mx = lax.axis_index("x")
    my = lax.axis_index("y")
    edge_x = jnp.where(mx == 0, x[m - 1 : m, :], x[0:1, :])
    edge_y = jnp.where(my == 0, x[:, n - 1 : n], x[:, 0:1])

    def body(
        x_hbm,
        ex_ref,
        ey_ref,
        out_hbm,
        slab_in,
        slab_out,
        halo_r,
        halo_c,
        send_sems,
        recv_sems,
        in_sems,
        out_sems,
    ):
        mx_ = lax.axis_index("x")
        my_ = lax.axis_index("y")
        xnbr = (1 - mx_, my_)
        ynbr = (mx_, 1 - my_)

        if not DEBUG_NO_RDMA:
            barrier = pltpu.get_barrier_semaphore()
            pl.semaphore_signal(
                barrier, inc=1, device_id=xnbr, device_id_type=pl.DeviceIdType.MESH
            )
            pl.semaphore_signal(
                barrier, inc=1, device_id=ynbr, device_id_type=pl.DeviceIdType.MESH
            )
            pl.semaphore_wait(barrier, 2)

            rdma_x = pltpu.make_async_remote_copy(
                src_ref=ex_ref,
                dst_ref=halo_r,
                send_sem=send_sems.at[0],
                recv_sem=recv_sems.at[0],
                device_id=xnbr,
                device_id_type=pl.DeviceIdType.MESH,
            )
            rdma_x.start()
            rdma_y = pltpu.make_async_remote_copy(
                src_ref=ey_ref,
                dst_ref=halo_c,
                send_sem=send_sems.at[1],
                recv_sem=recv_sems.at[1],
                device_id=ynbr,
                device_id_type=pl.DeviceIdType.MESH,
            )
            rdma_y.start()

        def make_in(c):
            buf = c % 2
            r0 = c * B
            lo = max(r0 - PAD, 0)
            hi = min(r0 + B + PAD, m)
            dlo = lo - (r0 - PAD)
            return pltpu.make_async_copy(
                x_hbm.at[lo:hi, :],
                slab_in.at[buf, dlo : dlo + (hi - lo), :],
                in_sems.at[buf],
            )

        def make_out(c):
            buf = c % 2
            r0 = c * B
            return pltpu.make_async_copy(
                slab_out.at[buf], out_hbm.at[r0 : r0 + B, :], out_sems.at[buf]
            )

        in_copies = {0: make_in(0)}
        in_copies[0].start()
        out_copies = {}

        if not DEBUG_NO_RDMA:
            rdma_x.wait()
            rdma_y.wait()

        for c in range(C):
            buf = c % 2
            if c + 1 < C:
                in_copies[c + 1] = make_in(c + 1)
                in_copies[c + 1].start()
            in_copies[c].wait()
            if c == 0:
                slab_in[buf, PAD - 1 : PAD, :] = halo_r[0:1, :]
            if c == C - 1:
                slab_in[buf, PAD + B : PAD + B + 1, :] = halo_r[0:1, :]
            if c >= 2:
                out_copies[c - 2].wait()

            r0 = c * B
            ct, cb = PAD, PAD + B
            slab_out[buf, :, 1 : n - 1] = 0.5 * slab_in[
                buf, ct:cb, 1 : n - 1
            ] + 0.125 * (
                slab_in[buf, ct - 1 : cb - 1, 1 : n - 1]
                + slab_in[buf, ct + 1 : cb + 1, 1 : n - 1]
                + slab_in[buf, ct:cb, 0 : n - 2]
                + slab_in[buf, ct:cb, 2:n]
            )
            col0 = 0.5 * slab_in[buf, ct:cb, 0:1] + 0.125 * (
                slab_in[buf, ct - 1 : cb - 1, 0:1]
                + slab_in[buf, ct + 1 : cb + 1, 0:1]
                + halo_c[r0 : r0 + B, 0:1]
                + slab_in[buf, ct:cb, 1:2]
            )
            slab_out[buf, :, 0:1] = jnp.where(
                my_ == 0, slab_in[buf, ct:cb, 0:1], col0
            )
            coln = 0.5 * slab_in[buf, ct:cb, n - 1 : n] + 0.125 * (
                slab_in[buf, ct - 1 : cb - 1, n - 1 : n]
                + slab_in[buf, ct + 1 : cb + 1, n - 1 : n]
                + slab_in[buf, ct:cb, n - 2 : n - 1]
                + halo_c[r0 : r0 + B, 0:1]
            )
            slab_out[buf, :, n - 1 : n] = jnp.where(
                my_ == 1, slab_in[buf, ct:cb, n - 1 : n], coln
            )
            if c == 0:
                slab_out[buf, 0:1, :] = jnp.where(
                    mx_ == 0, slab_in[buf, ct : ct + 1, :], slab_out[buf, 0:1, :]
                )
            if c == C - 1:
                slab_out[buf, B - 1 : B, :] = jnp.where(
                    mx_ == 1,
                    slab_in[buf, cb - 1 : cb, :],
                    slab_out[buf, B - 1 : B, :],
                )

            out_copies[c] = make_out(c)
            out_copies[c].start()

        out_copies[C - 2].wait()
        out_copies[C - 1].wait()

    return pl.pallas_call(
        body,
        out_shape=jax.ShapeDtypeStruct((m, n), x.dtype),
        in_specs=[
            pl.BlockSpec(memory_space=pl.ANY),
            pl.BlockSpec(memory_space=pltpu.VMEM),
            pl.BlockSpec(memory_space=pltpu.VMEM),
        ],
        out_specs=pl.BlockSpec(memory_space=pl.ANY),
        scratch_shapes=[
            pltpu.VMEM((2, B + 2 * PAD, n), x.dtype),
            pltpu.VMEM((2, B, n), x.dtype),
            pltpu.VMEM((1, n), x.dtype),
            pltpu.VMEM((m, 1), x.dtype),
            pltpu.SemaphoreType.DMA((2,)),
            pltpu.SemaphoreType.DMA((2,)),
            pltpu.SemaphoreType.DMA((2,)),
            pltpu.SemaphoreType.DMA((2,)),
        ],
        compiler_params=pltpu.CompilerParams(collective_id=0),
    )(x, edge_x, edge_y)
